# baseline (speedup 1.0000x reference)
import functools

import jax
import jax.numpy as jnp
import numpy as np
from jax import lax
from jax.experimental import pallas as pl
from jax.experimental.pallas import tpu as pltpu
from jax.experimental.pallas import tpu_sc as plsc

INPUT_DIM = 128
NUM_CLASSES = 64
NUM_TREES = 100
TREE_DEPTH = 5
NUM_USED_NODES = 16
NUM_LEAVES = 32
BATCH = 4096

SAMPLE_BLOCK = 512
TC_GRID = BATCH // SAMPLE_BLOCK

NUM_CORES = 2
NUM_SUBCORES = 16
NUM_WORKERS = NUM_CORES * NUM_SUBCORES
ROWS_PER_WORKER = BATCH // NUM_WORKERS
NUM_SLOTS = 16
LANES = 16
GROUPS = ROWS_PER_WORKER // LANES
CLASS_CHUNKS = NUM_CLASSES // LANES

_FLAT_NODES = NUM_TREES * NUM_USED_NODES
_WORKERS_PER_BLOCK = SAMPLE_BLOCK // ROWS_PER_WORKER


def _route_kernel(x_ref, w_ref, b_ref, p_ref, lv_ref, out_ref, lv_out):
    x = x_ref[...]
    w = w_ref[...]
    x_hi = x.astype(jnp.bfloat16)
    w_hi = w.astype(jnp.bfloat16)
    x_lo = (x - x_hi.astype(jnp.float32)).astype(jnp.bfloat16)
    w_lo = (w - w_hi.astype(jnp.float32)).astype(jnp.bfloat16)
    dims = (((1,), (1,)), ((), ()))

    def mm(a, bb):
        return lax.dot_general(a, bb, dims,
                               precision=lax.Precision.DEFAULT,
                               preferred_element_type=jnp.float32)

    logits = mm(x_hi, w_hi) + (mm(x_hi, w_lo) + mm(x_lo, w_hi))
    bits = jnp.where(logits + b_ref[...] <= 0.0, 1.0, 0.0)
    mask_f = lax.dot_general(
        bits, p_ref[...], (((1,), (0,)), ((), ())),
        precision=lax.Precision.DEFAULT)
    m = jnp.transpose(mask_f.astype(jnp.int32))
    idx = jnp.zeros_like(m)
    for _ in range(TREE_DEPTH):
        bit = lax.shift_right_logical(m, idx) & 1
        idx = idx + idx + bit
    tree = lax.broadcasted_iota(jnp.int32, m.shape, 0)
    ids = idx + tree * NUM_LEAVES
    out_ref[...] = jnp.transpose(
        ids.reshape(NUM_TREES, _WORKERS_PER_BLOCK, ROWS_PER_WORKER),
        (1, 0, 2))
    @pl.when(pl.program_id(0) == 0)
    def _():
        lv_out[...] = lv_ref[...].astype(jnp.bfloat16)


def _compute_leaf_ids(x, w_flat, b_flat, pack, leaves_flat):
    return pl.pallas_call(
        _route_kernel,
        grid=(TC_GRID,),
        in_specs=[
            pl.BlockSpec((SAMPLE_BLOCK, INPUT_DIM), lambda i: (i, 0)),
            pl.BlockSpec((_FLAT_NODES, INPUT_DIM), lambda i: (0, 0)),
            pl.BlockSpec((1, _FLAT_NODES), lambda i: (0, 0)),
            pl.BlockSpec((_FLAT_NODES, NUM_TREES), lambda i: (0, 0)),
            pl.BlockSpec((NUM_TREES * NUM_LEAVES, NUM_CLASSES),
                         lambda i: (0, 0)),
        ],
        out_specs=[
            pl.BlockSpec(
                (_WORKERS_PER_BLOCK, NUM_TREES, ROWS_PER_WORKER),
                lambda i: (i, 0, 0)),
            pl.BlockSpec((NUM_TREES * NUM_LEAVES, NUM_CLASSES),
                         lambda i: (0, 0)),
        ],
        out_shape=[
            jax.ShapeDtypeStruct(
                (NUM_WORKERS, NUM_TREES, ROWS_PER_WORKER), jnp.int32),
            jax.ShapeDtypeStruct(
                (NUM_TREES * NUM_LEAVES, NUM_CLASSES), jnp.bfloat16),
        ],
    )(x, w_flat, b_flat, pack, leaves_flat)


def _sc_body(ids_hbm, leaves_hbm, out_hbm, ids_v, acc_v, sem0):
    wid = lax.axis_index("s") * NUM_CORES + lax.axis_index("c")
    base = wid * ROWS_PER_WORKER

    pltpu.sync_copy(ids_hbm.at[wid], ids_v)

    for t in range(NUM_SLOTS):
        pltpu.async_copy(
            leaves_hbm.at[ids_v.at[t]], acc_v.at[t], sem0, add=False)
    for t in range(NUM_SLOTS):
        pltpu.make_async_copy(
            leaves_hbm.at[ids_v.at[0]], acc_v.at[0], sem0).wait()

    def tree_body(t, carry):
        pltpu.async_copy(
            leaves_hbm.at[ids_v.at[t]],
            acc_v.at[lax.rem(t, NUM_SLOTS)], sem0, add=True)
        return carry

    lax.fori_loop(NUM_SLOTS, NUM_TREES, tree_body, 0)

    def drain(t, carry):
        pltpu.make_async_copy(
            leaves_hbm.at[ids_v.at[0]], acc_v.at[0], sem0).wait()
        return carry

    lax.fori_loop(NUM_SLOTS, NUM_TREES, drain, 0)

    pltpu.sync_copy(acc_v,
                    out_hbm.at[:, pl.ds(base, ROWS_PER_WORKER), :])


@functools.cache
def _sc_forest():
    return pl.kernel(
        _sc_body,
        out_type=jax.ShapeDtypeStruct((NUM_SLOTS, BATCH, NUM_CLASSES),
                                      jnp.bfloat16),
        mesh=plsc.VectorSubcoreMesh(
            core_axis_name="c", subcore_axis_name="s",
            num_cores=NUM_CORES, num_subcores=NUM_SUBCORES),
        compiler_params=pltpu.CompilerParams(
            needs_layout_passes=False, use_tc_tiling_on_sc=False),
        scratch_types=[
            pltpu.VMEM((NUM_TREES, ROWS_PER_WORKER), jnp.int32),
            pltpu.VMEM((NUM_SLOTS, ROWS_PER_WORKER, NUM_CLASSES),
                       jnp.bfloat16),
            pltpu.SemaphoreType.DMA,
        ],
    )


def _reduce_kernel(p_ref, out_ref):
    out_ref[...] = (jnp.sum(p_ref[...].astype(jnp.float32), axis=0)
                    * (1.0 / NUM_TREES))


def _combine_partials(partials):
    return pl.pallas_call(
        _reduce_kernel,
        grid=(TC_GRID,),
        in_specs=[
            pl.BlockSpec((NUM_SLOTS, SAMPLE_BLOCK, NUM_CLASSES),
                         lambda i: (0, i, 0)),
        ],
        out_specs=pl.BlockSpec((SAMPLE_BLOCK, NUM_CLASSES), lambda i: (i, 0)),
        out_shape=jax.ShapeDtypeStruct((BATCH, NUM_CLASSES), jnp.float32),
    )(partials)


def _pack_matrix():
    p = np.zeros((_FLAT_NODES, NUM_TREES), np.float32)
    for t in range(NUM_TREES):
        for n in range(NUM_USED_NODES):
            p[t * NUM_USED_NODES + n, t] = float(1 << n)
    return jnp.asarray(p)


def kernel(x, W, b, leaves):
    w_flat = W[:, :NUM_USED_NODES, :].reshape(_FLAT_NODES, INPUT_DIM)
    b_flat = b[:, :NUM_USED_NODES].reshape(1, _FLAT_NODES)
    ids, leaves_bf = _compute_leaf_ids(
        x, w_flat, b_flat, _pack_matrix(),
        leaves.reshape(NUM_TREES * NUM_LEAVES, NUM_CLASSES))
    partials = _sc_forest()(ids, leaves_bf)
    return _combine_partials(partials)


# speedup vs baseline: 1.5490x; 1.5490x over previous
import functools

import jax
import jax.numpy as jnp
import numpy as np
from jax import lax
from jax.experimental import pallas as pl
from jax.experimental.pallas import tpu as pltpu
from jax.experimental.pallas import tpu_sc as plsc

INPUT_DIM = 128
NUM_CLASSES = 64
NUM_TREES = 100
TREE_DEPTH = 5
NUM_USED_NODES = 16
NUM_LEAVES = 32
BATCH = 4096

SAMPLE_BLOCK = 512
TC_GRID = BATCH // SAMPLE_BLOCK

NUM_CORES = 2
NUM_SUBCORES = 16
NUM_WORKERS = NUM_CORES * NUM_SUBCORES
ROWS_PER_WORKER = BATCH // NUM_WORKERS
NUM_SLOTS = 16
LANES = 16
GROUPS = ROWS_PER_WORKER // LANES
CLASS_CHUNKS = NUM_CLASSES // LANES

_FLAT_NODES = NUM_TREES * NUM_USED_NODES
_WORKERS_PER_BLOCK = SAMPLE_BLOCK // ROWS_PER_WORKER


def _route_kernel(x_ref, w_ref, b_ref, p_ref, out_ref):
    x = x_ref[...]
    w = w_ref[...]
    x_hi = x.astype(jnp.bfloat16)
    w_hi = w.astype(jnp.bfloat16)
    x_lo = (x - x_hi.astype(jnp.float32)).astype(jnp.bfloat16)
    w_lo = (w - w_hi.astype(jnp.float32)).astype(jnp.bfloat16)
    dims = (((1,), (1,)), ((), ()))

    def mm(a, bb):
        return lax.dot_general(a, bb, dims,
                               precision=lax.Precision.DEFAULT,
                               preferred_element_type=jnp.float32)

    logits = mm(x_hi, w_hi) + (mm(x_hi, w_lo) + mm(x_lo, w_hi))
    bits = jnp.where(logits + b_ref[...] <= 0.0, 1.0, 0.0)
    mask_f = lax.dot_general(
        bits, p_ref[...], (((1,), (0,)), ((), ())),
        precision=lax.Precision.DEFAULT)
    m = jnp.transpose(mask_f.astype(jnp.int32))
    idx = jnp.zeros_like(m)
    for _ in range(TREE_DEPTH):
        bit = lax.shift_right_logical(m, idx) & 1
        idx = idx + idx + bit
    tree = lax.broadcasted_iota(jnp.int32, m.shape, 0)
    ids = idx + tree * NUM_LEAVES
    out_ref[...] = jnp.transpose(
        ids.reshape(NUM_TREES, _WORKERS_PER_BLOCK, ROWS_PER_WORKER),
        (1, 0, 2))


def _compute_leaf_ids(x, w_flat, b_flat, pack):
    return pl.pallas_call(
        _route_kernel,
        grid=(TC_GRID,),
        in_specs=[
            pl.BlockSpec((SAMPLE_BLOCK, INPUT_DIM), lambda i: (i, 0)),
            pl.BlockSpec((_FLAT_NODES, INPUT_DIM), lambda i: (0, 0)),
            pl.BlockSpec((1, _FLAT_NODES), lambda i: (0, 0)),
            pl.BlockSpec((_FLAT_NODES, NUM_TREES), lambda i: (0, 0)),
        ],
        out_specs=pl.BlockSpec(
            (_WORKERS_PER_BLOCK, NUM_TREES, ROWS_PER_WORKER),
            lambda i: (i, 0, 0)),
        out_shape=jax.ShapeDtypeStruct(
            (NUM_WORKERS, NUM_TREES, ROWS_PER_WORKER), jnp.int32),
    )(x, w_flat, b_flat, pack)


def _sc_body(ids_hbm, leaves_hbm, out_hbm, ids_v, acc_v, final_v, sem0):
    wid = lax.axis_index("s") * NUM_CORES + lax.axis_index("c")
    base = wid * ROWS_PER_WORKER

    pltpu.sync_copy(ids_hbm.at[wid], ids_v)

    for t in range(NUM_SLOTS):
        pltpu.async_copy(
            leaves_hbm.at[ids_v.at[t]], acc_v.at[t], sem0, add=False)
    for t in range(NUM_SLOTS):
        pltpu.make_async_copy(
            leaves_hbm.at[ids_v.at[0]], acc_v.at[0], sem0).wait()

    def tree_body(t, carry):
        pltpu.async_copy(
            leaves_hbm.at[ids_v.at[t]],
            acc_v.at[lax.rem(t, NUM_SLOTS)], sem0, add=True)
        return carry

    lax.fori_loop(NUM_SLOTS, NUM_TREES, tree_body, 0)

    def drain(t, carry):
        pltpu.make_async_copy(
            leaves_hbm.at[ids_v.at[0]], acc_v.at[0], sem0).wait()
        return carry

    lax.fori_loop(NUM_SLOTS, NUM_TREES, drain, 0)

    lane_iota = lax.iota(jnp.int32, LANES)
    scale = jnp.full((LANES,), 1.0 / NUM_TREES, jnp.float32)
    hi_mask = jnp.full((LANES,), jnp.uint32(0xFFFF0000))

    def combine(r, carry):
        for ch in range(NUM_CLASSES // 32):
            esum = jnp.zeros((LANES,), jnp.float32)
            osum = jnp.zeros((LANES,), jnp.float32)
            for k in range(NUM_SLOTS):
                w = plsc.bitcast(
                    acc_v[k, r, pl.ds(ch * 32, 32)], jnp.uint32)
                esum = esum + plsc.bitcast(
                    lax.shift_left(w, jnp.uint32(16)), jnp.float32)
                osum = osum + plsc.bitcast(w & hi_mask, jnp.float32)
            col = ch * 32 + 2 * lane_iota
            fbase = r * NUM_CLASSES + col
            plsc.store_scatter(final_v, [fbase], esum * scale)
            plsc.store_scatter(final_v, [fbase + 1], osum * scale)
        return carry

    lax.fori_loop(0, ROWS_PER_WORKER, combine, 0)
    pltpu.sync_copy(
        final_v,
        out_hbm.at[pl.ds(base * NUM_CLASSES,
                         ROWS_PER_WORKER * NUM_CLASSES)])


@functools.cache
def _sc_forest():
    return pl.kernel(
        _sc_body,
        out_type=jax.ShapeDtypeStruct((BATCH * NUM_CLASSES,), jnp.float32),
        mesh=plsc.VectorSubcoreMesh(
            core_axis_name="c", subcore_axis_name="s",
            num_cores=NUM_CORES, num_subcores=NUM_SUBCORES),
        compiler_params=pltpu.CompilerParams(
            needs_layout_passes=False, use_tc_tiling_on_sc=False),
        scratch_types=[
            pltpu.VMEM((NUM_TREES, ROWS_PER_WORKER), jnp.int32),
            pltpu.VMEM((NUM_SLOTS, ROWS_PER_WORKER, NUM_CLASSES),
                       jnp.bfloat16),
            pltpu.VMEM((ROWS_PER_WORKER * NUM_CLASSES,), jnp.float32),
            pltpu.SemaphoreType.DMA,
        ],
    )


def _pack_matrix():
    p = np.zeros((_FLAT_NODES, NUM_TREES), np.float32)
    for t in range(NUM_TREES):
        for n in range(NUM_USED_NODES):
            p[t * NUM_USED_NODES + n, t] = float(1 << n)
    return jnp.asarray(p)


def kernel(x, W, b, leaves):
    w_flat = W[:, :NUM_USED_NODES, :].reshape(_FLAT_NODES, INPUT_DIM)
    b_flat = b[:, :NUM_USED_NODES].reshape(1, _FLAT_NODES)
    ids = _compute_leaf_ids(x, w_flat, b_flat, _pack_matrix())
    leaves_flat = leaves.astype(jnp.bfloat16).reshape(
        NUM_TREES * NUM_LEAVES, NUM_CLASSES)
    out = _sc_forest()(ids, leaves_flat)
    return out.reshape(BATCH, NUM_CLASSES)
